# baseline (speedup 1.0000x reference)
import jax
import jax.numpy as jnp
from jax.experimental import pallas as pl
from jax.experimental.pallas import tpu as pltpu

N0 = 8192
N1 = 8192
D_EDGE = 128
D_OUT = 128

_BM = 256
_NSTEP = N0 // (2 * _BM)


def _fused_kernel(pw_ref, da_ref, db_ref, x1_ref, wt_ref, b_ref,
                  oa_ref, ob_ref, h_ref):
    i = pl.program_id(0)

    @pl.when(i == 0)
    def _():
        h_ref[...] = jnp.dot(
            x1_ref[...], wt_ref[...],
            precision=jax.lax.Precision.DEFAULT,
            preferred_element_type=jnp.float32,
        ) + b_ref[...]

    w = pw_ref[0]
    h = h_ref[...]
    acc_a = jnp.dot(da_ref[...], h, precision=jax.lax.Precision.DEFAULT,
                    preferred_element_type=jnp.float32)
    oa_ref[...] = jnp.where(acc_a >= 0, acc_a, w * acc_a)
    acc_b = jnp.dot(db_ref[...], h, precision=jax.lax.Precision.DEFAULT,
                    preferred_element_type=jnp.float32)
    ob_ref[...] = jnp.where(acc_b >= 0, acc_b, w * acc_b)


def kernel(X0, X1, X2, L0, L1, L2, B2D3, D2B1TD1inv, D1invB1, B2TD2inv, W_e2n, b_e2n, prelu_w):
    grid = (_NSTEP,)
    half = N0 // 2
    ya, yb = pl.pallas_call(
        _fused_kernel,
        grid=grid,
        in_specs=[
            pl.BlockSpec(memory_space=pltpu.SMEM),
            pl.BlockSpec((_BM, N1), lambda i: (i, 0)),
            pl.BlockSpec((_BM, N1), lambda i: (i + _NSTEP, 0)),
            pl.BlockSpec((N1, D_EDGE), lambda i: (0, 0)),
            pl.BlockSpec((D_EDGE, D_OUT), lambda i: (0, 0)),
            pl.BlockSpec((1, D_OUT), lambda i: (0, 0)),
        ],
        out_specs=[
            pl.BlockSpec((_BM, D_OUT), lambda i: (i, 0)),
            pl.BlockSpec((_BM, D_OUT), lambda i: (i, 0)),
        ],
        out_shape=[
            jax.ShapeDtypeStruct((half, D_OUT), jnp.float32),
            jax.ShapeDtypeStruct((half, D_OUT), jnp.float32),
        ],
        scratch_shapes=[pltpu.VMEM((N1, D_OUT), jnp.float32)],
        compiler_params=pltpu.CompilerParams(
            dimension_semantics=("arbitrary",),
        ),
    )(prelu_w, D1invB1, D1invB1, X1, W_e2n.T, b_e2n.reshape(1, D_OUT))
    return jnp.concatenate([ya, yb], axis=0)


# speedup vs baseline: 1.0513x; 1.0513x over previous
import jax
import jax.numpy as jnp
from jax.experimental import pallas as pl
from jax.experimental.pallas import tpu as pltpu

N0 = 8192
N1 = 8192
D_EDGE = 128
D_OUT = 128

_BM = 256


def _fused_kernel(pw_ref, d_ref, x1_ref, wt_ref, b_ref, o_ref, h_ref):
    i = pl.program_id(0)

    @pl.when(i == 0)
    def _():
        h_ref[...] = jnp.dot(
            x1_ref[...], wt_ref[...],
            precision=jax.lax.Precision.DEFAULT,
            preferred_element_type=jnp.float32,
        ) + b_ref[...]

    acc = jnp.dot(
        d_ref[...], h_ref[...],
        precision=jax.lax.Precision.DEFAULT,
        preferred_element_type=jnp.float32,
    )
    w = pw_ref[0]
    o_ref[...] = jnp.where(acc >= 0, acc, w * acc)


def kernel(X0, X1, X2, L0, L1, L2, B2D3, D2B1TD1inv, D1invB1, B2TD2inv, W_e2n, b_e2n, prelu_w):
    grid = (N0 // _BM,)
    y = pl.pallas_call(
        _fused_kernel,
        grid=grid,
        in_specs=[
            pl.BlockSpec(memory_space=pltpu.SMEM),
            pl.BlockSpec((_BM, N1), lambda i: (i, 0)),
            pl.BlockSpec((N1, D_EDGE), lambda i: (0, 0)),
            pl.BlockSpec((D_EDGE, D_OUT), lambda i: (0, 0)),
            pl.BlockSpec((1, D_OUT), lambda i: (0, 0)),
        ],
        out_specs=pl.BlockSpec((_BM, D_OUT), lambda i: (i, 0)),
        out_shape=jax.ShapeDtypeStruct((N0, D_OUT), jnp.float32),
        scratch_shapes=[pltpu.VMEM((N1, D_OUT), jnp.float32)],
        compiler_params=pltpu.CompilerParams(
            dimension_semantics=("arbitrary",),
        ),
    )(prelu_w, D1invB1, X1, W_e2n.T, b_e2n.reshape(1, D_OUT))
    return y


# speedup vs baseline: 1.0549x; 1.0034x over previous
import jax
import jax.numpy as jnp
from jax.experimental import pallas as pl
from jax.experimental.pallas import tpu as pltpu

N0 = 8192
N1 = 8192
D_EDGE = 128
D_OUT = 128

_BM = 256


def _fused_kernel(pw_ref, d_ref, x1_ref, wt_ref, b_ref, o_ref, h_ref):
    i = pl.program_id(0)

    @pl.when(i == 0)
    def _():
        h_ref[...] = jnp.dot(
            x1_ref[...], wt_ref[...],
            precision=jax.lax.Precision.DEFAULT,
            preferred_element_type=jnp.float32,
        ) + b_ref[...]

    acc = jnp.dot(
        d_ref[...], h_ref[...],
        precision=jax.lax.Precision.DEFAULT,
        preferred_element_type=jnp.float32,
    )
    w = pw_ref[0]
    o_ref[...] = jnp.where(acc >= 0, acc, w * acc)


def kernel(X0, X1, X2, L0, L1, L2, B2D3, D2B1TD1inv, D1invB1, B2TD2inv, W_e2n, b_e2n, prelu_w):
    grid = (N0 // _BM,)
    y = pl.pallas_call(
        _fused_kernel,
        grid=grid,
        in_specs=[
            pl.BlockSpec(memory_space=pltpu.SMEM),
            pl.BlockSpec((_BM, N1), lambda i: (i, 0)),
            pl.BlockSpec((N1, D_EDGE), lambda i: (0, 0)),
            pl.BlockSpec((D_EDGE, D_OUT), lambda i: (0, 0)),
            pl.BlockSpec((1, D_OUT), lambda i: (0, 0)),
        ],
        out_specs=pl.BlockSpec((_BM, D_OUT), lambda i: (i, 0)),
        out_shape=jax.ShapeDtypeStruct((N0, D_OUT), jnp.float32),
        scratch_shapes=[pltpu.VMEM((N1, D_OUT), jnp.float32)],
        compiler_params=pltpu.CompilerParams(
            dimension_semantics=("parallel",),
        ),
    )(prelu_w, D1invB1, X1, W_e2n.T, b_e2n.reshape(1, D_OUT))
    return y
